# baseline (speedup 1.0000x reference)
import functools

import jax
import jax.numpy as jnp
from jax import lax
from jax.experimental import pallas as pl
from jax.experimental.pallas import tpu as pltpu, tpu_sc as plsc

_NNK = 16
_N = 2048
_B = 2
_NW = 32
_ROWS_PER_W = _B * _N // _NW
_ROWS = 1024


def _sc_knn2(yt_flat):
    mesh = plsc.VectorSubcoreMesh(core_axis_name="c", subcore_axis_name="s")

    @functools.partial(
        pl.kernel,
        mesh=mesh,
        compiler_params=pltpu.CompilerParams(needs_layout_passes=False),
        out_type=jax.ShapeDtypeStruct((_B * _N * _NNK,), jnp.float32),
        scratch_types=[
            pltpu.VMEM((_N,), jnp.float32),
            pltpu.VMEM((_N,), jnp.float32),
            pltpu.VMEM((_N,), jnp.float32),
            pltpu.VMEM((_NNK,), jnp.float32),
        ],
    )
    def k(yt_hbm, out_hbm, cx_v, cy_v, cz_v, row_v):
        wid = lax.axis_index("s") * 2 + lax.axis_index("c")
        batch = wid // (_NW // _B)
        boff = batch * 3 * _N
        pltpu.sync_copy(yt_hbm.at[pl.ds(boff, _N)], cx_v)
        pltpu.sync_copy(yt_hbm.at[pl.ds(boff + _N, _N)], cy_v)
        pltpu.sync_copy(yt_hbm.at[pl.ds(boff + 2 * _N, _N)], cz_v)

        def row_body(rr, carry):
            g = wid * _ROWS_PER_W + rr
            r = g % _N
            base = (r // 16) * 16
            lane = jnp.full((16, 1), r % 16, jnp.int32)
            dnums = lax.GatherDimensionNumbers(
                offset_dims=(), collapsed_slice_dims=(0,), start_index_map=(0,)
            )

            def bcast(ref):
                return lax.gather(
                    ref[pl.ds(base, 16)],
                    lane,
                    dnums,
                    (1,),
                    mode=lax.GatherScatterMode.PROMISE_IN_BOUNDS,
                )

            xr = bcast(cx_v)
            yr = bcast(cy_v)
            zr = bcast(cz_v)

            def chunk_body(j, top):
                cx = cx_v[pl.ds(j * 16, 16)]
                cy = cy_v[pl.ds(j * 16, 16)]
                cz = cz_v[pl.ds(j * 16, 16)]
                dx = cx - xr
                dy = cy - yr
                dz = cz - zr
                d2 = dx * dx + dy * dy + dz * dz
                c, _ = plsc.sort_key_val(d2, d2)
                merged = jnp.minimum(top, jnp.flip(c, 0))
                out, _ = plsc.sort_key_val(merged, merged)
                return out

            top0 = jnp.full((_NNK,), jnp.float32(3e38), jnp.float32)
            top = lax.fori_loop(0, _N // 16, chunk_body, top0)
            row_v[...] = top
            pltpu.sync_copy(row_v, out_hbm.at[pl.ds(g * _NNK, _NNK)])
            return carry

        lax.fori_loop(0, _ROWS_PER_W, row_body, 0)

    return k(yt_flat)


def _tc_body(yp_nat, yt_cols, knn_out, mincol_out, mincol_acc):
    i = pl.program_id(1)
    ni = pl.num_programs(1)
    n = yp_nat.shape[1]

    yp3 = yp_nat[0]
    xt = yt_cols[0]

    inf = jnp.float32(jnp.inf)
    big = jnp.float32(3e38)

    a_t = None
    for c in range(3):
        d = yp3[:, c : c + 1] - xt[c : c + 1, :]
        a_t = d * d if a_t is None else a_t + d * d

    colmin = jnp.min(a_t, axis=1, keepdims=True)

    @pl.when(i == 0)
    def _cm0():
        mincol_acc[...] = colmin

    @pl.when(i != 0)
    def _cm1():
        mincol_acc[...] = jnp.minimum(mincol_acc[...], colmin)

    @pl.when(i == ni - 1)
    def _fin():
        mincol_out[...] = mincol_acc[...].reshape(1, 1, n)

    iota = jax.lax.broadcasted_iota(jnp.int32, (n, _ROWS), 0).astype(jnp.float32)

    v = a_t
    for k in range(_NNK):
        m = jnp.min(v, axis=0, keepdims=True)
        t = jnp.where(v == m, iota, big)
        idx = jnp.min(t, axis=0, keepdims=True)
        v = jnp.where(t == idx, inf, v)
        knn_out[k : k + 1, :] = m


def _tc_knn1(y_pred, yt_cols):
    ni = _N // _ROWS
    return pl.pallas_call(
        _tc_body,
        grid=(_B, ni),
        in_specs=[
            pl.BlockSpec((1, _N, 3), lambda b, i: (b, 0, 0)),
            pl.BlockSpec((1, 3, _ROWS), lambda b, i: (b, 0, i)),
        ],
        out_specs=[
            pl.BlockSpec((_NNK, _ROWS), lambda b, i: (0, b * (_N // _ROWS) + i)),
            pl.BlockSpec((1, 1, _N), lambda b, i: (b, 0, 0)),
        ],
        out_shape=[
            jax.ShapeDtypeStruct((_NNK, _B * _N), jnp.float32),
            jax.ShapeDtypeStruct((_B, 1, _N), jnp.float32),
        ],
        scratch_shapes=[pltpu.VMEM((_N, 1), jnp.float32)],
    )(y_pred, yt_cols)


@jax.jit
def kernel(y_pred, y_true):
    yt_cols = jnp.transpose(y_true, (0, 2, 1))
    knn2_flat = _sc_knn2(yt_cols.reshape(-1))
    knn1, mincol_sq = _tc_knn1(y_pred, yt_cols)

    n_rows = _B * _N
    sq_a = jnp.sqrt(knn1)
    sq_b = jnp.sqrt(jnp.transpose(knn2_flat.reshape(n_rows, _NNK)))
    shape_loss = (
        jnp.sum(sq_a[0]) / n_rows + jnp.sum(jnp.sqrt(mincol_sq)) / n_rows
    ) * 0.5
    density_loss = jnp.mean(jnp.abs(sq_a - sq_b))
    data_loss = shape_loss + density_loss
    return (data_loss, shape_loss, density_loss)


# speedup vs baseline: 1.0011x; 1.0011x over previous
import functools

import jax
import jax.numpy as jnp
from jax import lax
from jax.experimental import pallas as pl
from jax.experimental.pallas import tpu as pltpu, tpu_sc as plsc

_NNK = 16
_N = 2048
_B = 2
_NW = 32
_ROWS_PER_W = _B * _N // _NW
_ROWS = 1024


def _sc_knn2(yt_flat):
    mesh = plsc.VectorSubcoreMesh(core_axis_name="c", subcore_axis_name="s")

    @functools.partial(
        pl.kernel,
        mesh=mesh,
        compiler_params=pltpu.CompilerParams(needs_layout_passes=False),
        out_type=jax.ShapeDtypeStruct((_B * _N * _NNK,), jnp.float32),
        scratch_types=[
            pltpu.VMEM((_N,), jnp.float32),
            pltpu.VMEM((_N,), jnp.float32),
            pltpu.VMEM((_N,), jnp.float32),
            pltpu.VMEM((_NNK,), jnp.float32),
        ],
    )
    def k(yt_hbm, out_hbm, cx_v, cy_v, cz_v, row_v):
        wid = lax.axis_index("s") * 2 + lax.axis_index("c")
        batch = wid // (_NW // _B)
        boff = batch * 3 * _N
        pltpu.sync_copy(yt_hbm.at[pl.ds(boff, _N)], cx_v)
        pltpu.sync_copy(yt_hbm.at[pl.ds(boff + _N, _N)], cy_v)
        pltpu.sync_copy(yt_hbm.at[pl.ds(boff + 2 * _N, _N)], cz_v)

        def row_body(rr, carry):
            g = wid * _ROWS_PER_W + rr
            r = g % _N
            base = (r // 16) * 16
            lane = jnp.full((16, 1), r % 16, jnp.int32)
            dnums = lax.GatherDimensionNumbers(
                offset_dims=(), collapsed_slice_dims=(0,), start_index_map=(0,)
            )

            def bcast(ref):
                return lax.gather(
                    ref[pl.ds(base, 16)],
                    lane,
                    dnums,
                    (1,),
                    mode=lax.GatherScatterMode.PROMISE_IN_BOUNDS,
                )

            xr = bcast(cx_v)
            yr = bcast(cy_v)
            zr = bcast(cz_v)

            def d2_of(jj):
                cx = cx_v[pl.ds(jj * 16, 16)]
                cy = cy_v[pl.ds(jj * 16, 16)]
                cz = cz_v[pl.ds(jj * 16, 16)]
                dx = cx - xr
                dy = cy - yr
                dz = cz - zr
                return dx * dx + dy * dy + dz * dz

            def chunk_body(j, top):
                d2a = d2_of(j * 2)
                d2b = d2_of(j * 2 + 1)
                ca, _ = plsc.sort_key_val(d2a, d2a)
                cb, _ = plsc.sort_key_val(d2b, d2b)
                m1 = jnp.minimum(top, jnp.flip(ca, 0))
                t1, _ = plsc.sort_key_val(m1, m1)
                m2 = jnp.minimum(t1, jnp.flip(cb, 0))
                t2, _ = plsc.sort_key_val(m2, m2)
                return t2

            top0 = jnp.full((_NNK,), jnp.float32(3e38), jnp.float32)
            top = lax.fori_loop(0, _N // 32, chunk_body, top0)
            row_v[...] = top
            pltpu.sync_copy(row_v, out_hbm.at[pl.ds(g * _NNK, _NNK)])
            return carry

        lax.fori_loop(0, _ROWS_PER_W, row_body, 0)

    return k(yt_flat)


def _tc_body(yp_nat, yt_cols, knn_out, mincol_out, mincol_acc):
    i = pl.program_id(1)
    ni = pl.num_programs(1)
    n = yp_nat.shape[1]

    yp3 = yp_nat[0]
    xt = yt_cols[0]

    inf = jnp.float32(jnp.inf)
    big = jnp.float32(3e38)

    a_t = None
    for c in range(3):
        d = yp3[:, c : c + 1] - xt[c : c + 1, :]
        a_t = d * d if a_t is None else a_t + d * d

    colmin = jnp.min(a_t, axis=1, keepdims=True)

    @pl.when(i == 0)
    def _cm0():
        mincol_acc[...] = colmin

    @pl.when(i != 0)
    def _cm1():
        mincol_acc[...] = jnp.minimum(mincol_acc[...], colmin)

    @pl.when(i == ni - 1)
    def _fin():
        mincol_out[...] = mincol_acc[...].reshape(1, 1, n)

    iota = jax.lax.broadcasted_iota(jnp.int32, (n, _ROWS), 0).astype(jnp.float32)

    v = a_t
    for k in range(_NNK):
        m = jnp.min(v, axis=0, keepdims=True)
        t = jnp.where(v == m, iota, big)
        idx = jnp.min(t, axis=0, keepdims=True)
        v = jnp.where(t == idx, inf, v)
        knn_out[k : k + 1, :] = m


def _tc_knn1(y_pred, yt_cols):
    ni = _N // _ROWS
    return pl.pallas_call(
        _tc_body,
        grid=(_B, ni),
        in_specs=[
            pl.BlockSpec((1, _N, 3), lambda b, i: (b, 0, 0)),
            pl.BlockSpec((1, 3, _ROWS), lambda b, i: (b, 0, i)),
        ],
        out_specs=[
            pl.BlockSpec((_NNK, _ROWS), lambda b, i: (0, b * (_N // _ROWS) + i)),
            pl.BlockSpec((1, 1, _N), lambda b, i: (b, 0, 0)),
        ],
        out_shape=[
            jax.ShapeDtypeStruct((_NNK, _B * _N), jnp.float32),
            jax.ShapeDtypeStruct((_B, 1, _N), jnp.float32),
        ],
        scratch_shapes=[pltpu.VMEM((_N, 1), jnp.float32)],
    )(y_pred, yt_cols)


@jax.jit
def kernel(y_pred, y_true):
    yt_cols = jnp.transpose(y_true, (0, 2, 1))
    knn2_flat = _sc_knn2(yt_cols.reshape(-1))
    knn1, mincol_sq = _tc_knn1(y_pred, yt_cols)

    n_rows = _B * _N
    sq_a = jnp.sqrt(knn1)
    sq_b = jnp.sqrt(jnp.transpose(knn2_flat.reshape(n_rows, _NNK)))
    shape_loss = (
        jnp.sum(sq_a[0]) / n_rows + jnp.sum(jnp.sqrt(mincol_sq)) / n_rows
    ) * 0.5
    density_loss = jnp.mean(jnp.abs(sq_a - sq_b))
    data_loss = shape_loss + density_loss
    return (data_loss, shape_loss, density_loss)
